# Initial kernel scaffold; baseline (speedup 1.0000x reference)
import jax
import jax.numpy as jnp
from jax.experimental import pallas as pl


def kernel(x, table):
    raise NotImplementedError("write your pallas kernel here")


# speedup vs baseline: 2.3867x; 2.3867x over previous
import functools

import jax
import jax.numpy as jnp
import numpy as np
from jax import lax
from jax.experimental import pallas as pl
from jax.experimental.pallas import tpu as pltpu
from jax.experimental.pallas import tpu_sc as plsc

INPUT_DIM = 100000
OUTPUT_DIM = 64
BATCH = 4096
SEQ = 200
HALF = SEQ // 2
SCALE = float(np.sqrt(np.float32(OUTPUT_DIM)))


def _pos_encoding(position, d_model):
    i = np.arange(d_model)[np.newaxis, :]
    angle_rates = 1 / np.power(10000, 2 * (i // 2) / np.float32(d_model))
    angle_rads = np.arange(position)[:, np.newaxis] * angle_rates
    angle_rads[:, 0::2] = np.sin(angle_rads[:, 0::2])
    angle_rads[:, 1::2] = np.cos(angle_rads[:, 1::2])
    return np.asarray(angle_rads, dtype=np.float32)


_POS = _pos_encoding(SEQ, OUTPUT_DIM)


@functools.cache
def _build_kernel(nc, ns):
    nw = nc * ns
    total_rows = BATCH * SEQ
    rows_per_w = total_rows // nw
    seqs_per_w = rows_per_w // SEQ
    chunks_per_w = rows_per_w // HALF

    mesh = plsc.VectorSubcoreMesh(
        core_axis_name="c", subcore_axis_name="s",
        num_cores=nc, num_subcores=ns)

    @functools.partial(
        pl.kernel,
        out_type=jax.ShapeDtypeStruct((total_rows, OUTPUT_DIM), jnp.float32),
        mesh=mesh,
        scratch_types=[
            pltpu.VMEM((chunks_per_w, HALF), jnp.int32),
            pltpu.VMEM((SEQ, OUTPUT_DIM), jnp.float32),
            pltpu.VMEM((SEQ, OUTPUT_DIM), jnp.float32),
            pltpu.SemaphoreType.DMA,
        ],
        compiler_params=pltpu.CompilerParams(use_tc_tiling_on_sc=False),
    )
    def emb_kernel(idx_hbm, table_hbm, pos_hbm, out_hbm, idx_v, pos_v, buf, gsem):
        wid = lax.axis_index("s") * nc + lax.axis_index("c")
        pltpu.sync_copy(idx_hbm.at[wid], idx_v)
        pltpu.sync_copy(pos_hbm, pos_v)
        base = wid * rows_per_w

        def seq_body(q, carry):
            cp0 = pltpu.async_copy(
                table_hbm.at[idx_v.at[2 * q]], buf.at[pl.ds(0, HALF)], gsem)
            cp1 = pltpu.async_copy(
                table_hbm.at[idx_v.at[2 * q + 1]], buf.at[pl.ds(HALF, HALF)], gsem)
            cp0.wait()
            cp1.wait()

            def row_body(r, c2):
                for c in range(OUTPUT_DIM // 16):
                    sl = pl.ds(c * 16, 16)
                    buf[r, sl] = buf[r, sl] * SCALE + pos_v[r, sl]
                return c2
            lax.fori_loop(0, SEQ, row_body, 0, unroll=2)

            pltpu.sync_copy(buf, out_hbm.at[pl.ds(base + q * SEQ, SEQ)])
            return carry

        lax.fori_loop(0, seqs_per_w, seq_body, 0)

    return emb_kernel


def kernel(x, table):
    info = plsc.get_sparse_core_info()
    nc, ns = info.num_cores, info.num_subcores
    nw = nc * ns
    idx = x.reshape(nw, (BATCH * SEQ) // nw // HALF, HALF)
    pos = jnp.asarray(_POS)
    out = _build_kernel(nc, ns)(idx, table, pos)
    return out.reshape(BATCH, SEQ, OUTPUT_DIM)


# speedup vs baseline: 2.6186x; 1.0972x over previous
import functools

import jax
import jax.numpy as jnp
import numpy as np
from jax import lax
from jax.experimental import pallas as pl
from jax.experimental.pallas import tpu as pltpu
from jax.experimental.pallas import tpu_sc as plsc

INPUT_DIM = 100000
OUTPUT_DIM = 64
BATCH = 4096
SEQ = 200
HALF = SEQ // 2
SCALE = float(np.sqrt(np.float32(OUTPUT_DIM)))


def _pos_encoding(position, d_model):
    i = np.arange(d_model)[np.newaxis, :]
    angle_rates = 1 / np.power(10000, 2 * (i // 2) / np.float32(d_model))
    angle_rads = np.arange(position)[:, np.newaxis] * angle_rates
    angle_rads[:, 0::2] = np.sin(angle_rads[:, 0::2])
    angle_rads[:, 1::2] = np.cos(angle_rads[:, 1::2])
    return np.asarray(angle_rads, dtype=np.float32)


_POS = _pos_encoding(SEQ, OUTPUT_DIM)


@functools.cache
def _build_kernel(nc, ns):
    nw = nc * ns
    total_rows = BATCH * SEQ
    rows_per_w = total_rows // nw
    seqs_per_w = rows_per_w // SEQ
    chunks_per_w = rows_per_w // HALF

    mesh = plsc.VectorSubcoreMesh(
        core_axis_name="c", subcore_axis_name="s",
        num_cores=nc, num_subcores=ns)

    @functools.partial(
        pl.kernel,
        out_type=jax.ShapeDtypeStruct((total_rows, OUTPUT_DIM), jnp.float32),
        mesh=mesh,
        scratch_types=[
            pltpu.VMEM((chunks_per_w, HALF), jnp.int32),
            pltpu.VMEM((SEQ, OUTPUT_DIM), jnp.float32),
            pltpu.VMEM((SEQ, OUTPUT_DIM), jnp.float32),
            pltpu.VMEM((SEQ, OUTPUT_DIM), jnp.float32),
            pltpu.VMEM((SEQ, OUTPUT_DIM), jnp.float32),
            pltpu.VMEM((SEQ, OUTPUT_DIM), jnp.float32),
            pltpu.SemaphoreType.DMA,
            pltpu.SemaphoreType.DMA,
            pltpu.SemaphoreType.DMA,
            pltpu.SemaphoreType.DMA,
        ],
        compiler_params=pltpu.CompilerParams(use_tc_tiling_on_sc=False),
    )
    def emb_kernel(idx_hbm, table_hbm, pos_hbm, out_hbm, idx_v, pos_v,
                   g0, g1, s0, s1, gsem0, gsem1, ssem0, ssem1):
        wid = lax.axis_index("s") * nc + lax.axis_index("c")
        pltpu.sync_copy(idx_hbm.at[wid], idx_v)
        pltpu.sync_copy(pos_hbm, pos_v)
        base = wid * rows_per_w
        gbufs, sbufs = (g0, g1), (s0, s1)
        gsems, ssems = (gsem0, gsem1), (ssem0, ssem1)

        def fire_gather(si, gb, gsem):
            pltpu.async_copy(
                table_hbm.at[idx_v.at[2 * si]], gb.at[pl.ds(0, HALF)], gsem)
            pltpu.async_copy(
                table_hbm.at[idx_v.at[2 * si + 1]], gb.at[pl.ds(HALF, HALF)], gsem)

        def wait_gather(si, gb, gsem):
            pltpu.make_async_copy(
                table_hbm.at[idx_v.at[2 * si]], gb.at[pl.ds(0, HALF)], gsem).wait()
            pltpu.make_async_copy(
                table_hbm.at[idx_v.at[2 * si + 1]], gb.at[pl.ds(HALF, HALF)], gsem).wait()

        fire_gather(0, g0, gsem0)
        fire_gather(1, g1, gsem1)

        def body(j, carry):
            q = 2 * j
            for b in range(2):
                si = q + b
                gb, sb, gsem, ssem = gbufs[b], sbufs[b], gsems[b], ssems[b]
                wait_gather(si, gb, gsem)

                @pl.when(si >= 2)
                def _():
                    pltpu.make_async_copy(
                        sb, out_hbm.at[pl.ds(base, SEQ)], ssem).wait()

                def row_body(r, c2):
                    for c in range(OUTPUT_DIM // 16):
                        sl = pl.ds(c * 16, 16)
                        sb[r, sl] = gb[r, sl] * SCALE + pos_v[r, sl]
                    return c2
                lax.fori_loop(0, SEQ, row_body, 0, unroll=4)

                pltpu.async_copy(sb, out_hbm.at[pl.ds(base + si * SEQ, SEQ)], ssem)

                @pl.when(si + 2 < seqs_per_w)
                def _():
                    fire_gather(si + 2, gb, gsem)
            return carry

        lax.fori_loop(0, seqs_per_w // 2, body, 0)
        pltpu.make_async_copy(s0, out_hbm.at[pl.ds(base, SEQ)], ssem0).wait()
        pltpu.make_async_copy(s1, out_hbm.at[pl.ds(base, SEQ)], ssem1).wait()

    return emb_kernel


def kernel(x, table):
    info = plsc.get_sparse_core_info()
    nc, ns = info.num_cores, info.num_subcores
    nw = nc * ns
    idx = x.reshape(nw, (BATCH * SEQ) // nw // HALF, HALF)
    pos = jnp.asarray(_POS)
    out = _build_kernel(nc, ns)(idx, table, pos)
    return out.reshape(BATCH, SEQ, OUTPUT_DIM)


# speedup vs baseline: 4.2473x; 1.6220x over previous
import functools

import jax
import jax.numpy as jnp
import numpy as np
from jax import lax
from jax.experimental import pallas as pl
from jax.experimental.pallas import tpu as pltpu
from jax.experimental.pallas import tpu_sc as plsc

INPUT_DIM = 100000
OUTPUT_DIM = 64
BATCH = 4096
SEQ = 200
HALF = SEQ // 2
SCALE = float(np.sqrt(np.float32(OUTPUT_DIM)))


def _pos_encoding(position, d_model):
    i = np.arange(d_model)[np.newaxis, :]
    angle_rates = 1 / np.power(10000, 2 * (i // 2) / np.float32(d_model))
    angle_rads = np.arange(position)[:, np.newaxis] * angle_rates
    angle_rads[:, 0::2] = np.sin(angle_rads[:, 0::2])
    angle_rads[:, 1::2] = np.cos(angle_rads[:, 1::2])
    return np.asarray(angle_rads, dtype=np.float32)


_POS = _pos_encoding(SEQ, OUTPUT_DIM)


@functools.cache
def _build_kernel(nc, ns):
    nw = nc * ns
    total_rows = BATCH * SEQ
    rows_per_w = total_rows // nw
    seqs_per_w = rows_per_w // SEQ
    chunks_per_w = rows_per_w // HALF

    mesh = plsc.VectorSubcoreMesh(
        core_axis_name="c", subcore_axis_name="s",
        num_cores=nc, num_subcores=ns)

    @functools.partial(
        pl.kernel,
        out_type=jax.ShapeDtypeStruct((BATCH, SEQ, OUTPUT_DIM), jnp.float32),
        mesh=mesh,
        scratch_types=[
            pltpu.VMEM((chunks_per_w, HALF), jnp.int32),
            pltpu.VMEM((SEQ, OUTPUT_DIM), jnp.float32),
            pltpu.VMEM((SEQ, OUTPUT_DIM), jnp.float32),
            pltpu.VMEM((SEQ, OUTPUT_DIM), jnp.float32),
            pltpu.VMEM((SEQ, OUTPUT_DIM), jnp.float32),
            pltpu.VMEM((SEQ, OUTPUT_DIM), jnp.float32),
            pltpu.SemaphoreType.DMA,
            pltpu.SemaphoreType.DMA,
            pltpu.SemaphoreType.DMA,
            pltpu.SemaphoreType.DMA,
        ],
        compiler_params=pltpu.CompilerParams(use_tc_tiling_on_sc=False),
    )
    def emb_kernel(idx_hbm, table_hbm, pos_hbm, out_hbm, idx_v, pos_v,
                   g0, g1, s0, s1, gsem0, gsem1, ssem0, ssem1):
        wid = lax.axis_index("s") * nc + lax.axis_index("c")
        pltpu.sync_copy(idx_hbm.at[wid], idx_v)
        pltpu.sync_copy(pos_hbm, pos_v)
        base = wid * seqs_per_w
        gbufs, sbufs = (g0, g1), (s0, s1)
        gsems, ssems = (gsem0, gsem1), (ssem0, ssem1)

        def fire_gather(si, gb, gsem):
            pltpu.async_copy(
                table_hbm.at[idx_v.at[2 * si]], gb.at[pl.ds(0, HALF)], gsem)
            pltpu.async_copy(
                table_hbm.at[idx_v.at[2 * si + 1]], gb.at[pl.ds(HALF, HALF)], gsem)

        def wait_gather(si, gb, gsem):
            pltpu.make_async_copy(
                table_hbm.at[idx_v.at[2 * si]], gb.at[pl.ds(0, HALF)], gsem).wait()
            pltpu.make_async_copy(
                table_hbm.at[idx_v.at[2 * si + 1]], gb.at[pl.ds(HALF, HALF)], gsem).wait()

        fire_gather(0, g0, gsem0)
        fire_gather(1, g1, gsem1)

        def body(j, carry):
            q = 2 * j
            for b in range(2):
                si = q + b
                gb, sb, gsem, ssem = gbufs[b], sbufs[b], gsems[b], ssems[b]
                wait_gather(si, gb, gsem)

                @pl.when(si >= 2)
                def _():
                    pltpu.make_async_copy(sb, out_hbm.at[base], ssem).wait()

                @functools.partial(plsc.parallel_loop, 0, SEQ, unroll=4)
                def _(r):
                    for c in range(OUTPUT_DIM // 16):
                        sl = pl.ds(c * 16, 16)
                        sb[r, sl] = gb[r, sl] * SCALE + pos_v[r, sl]

                pltpu.async_copy(sb, out_hbm.at[base + si], ssem)

                @pl.when(si + 2 < seqs_per_w)
                def _():
                    fire_gather(si + 2, gb, gsem)
            return carry

        lax.fori_loop(0, seqs_per_w // 2, body, 0)
        pltpu.make_async_copy(s0, out_hbm.at[base], ssem0).wait()
        pltpu.make_async_copy(s1, out_hbm.at[base], ssem1).wait()

    return emb_kernel


def kernel(x, table):
    info = plsc.get_sparse_core_info()
    nc, ns = info.num_cores, info.num_subcores
    nw = nc * ns
    idx = x.reshape(nw, (BATCH * SEQ) // nw // HALF, HALF)
    pos = jnp.asarray(_POS)
    return _build_kernel(nc, ns)(idx, table, pos)


# speedup vs baseline: 4.6752x; 1.1008x over previous
import functools

import jax
import jax.numpy as jnp
import numpy as np
from jax import lax
from jax.experimental import pallas as pl
from jax.experimental.pallas import tpu as pltpu
from jax.experimental.pallas import tpu_sc as plsc

INPUT_DIM = 100000
OUTPUT_DIM = 64
PAD_DIM = 128
BATCH = 4096
SEQ = 200
CHUNK = 40
NBUF = 4
SCALE = float(np.sqrt(np.float32(OUTPUT_DIM)))


def _pos_encoding(position, d_model):
    i = np.arange(d_model)[np.newaxis, :]
    angle_rates = 1 / np.power(10000, 2 * (i // 2) / np.float32(d_model))
    angle_rads = np.arange(position)[:, np.newaxis] * angle_rates
    angle_rads[:, 0::2] = np.sin(angle_rads[:, 0::2])
    angle_rads[:, 1::2] = np.cos(angle_rads[:, 1::2])
    return np.asarray(angle_rads, dtype=np.float32)


_POS = _pos_encoding(SEQ, OUTPUT_DIM)


@functools.cache
def _build_kernel(nc, ns):
    nw = nc * ns
    total_rows = BATCH * SEQ
    rows_per_w = total_rows // nw
    seqs_per_w = rows_per_w // SEQ
    chunks_per_w = rows_per_w // CHUNK
    cps = SEQ // CHUNK

    mesh = plsc.VectorSubcoreMesh(
        core_axis_name="c", subcore_axis_name="s",
        num_cores=nc, num_subcores=ns)

    @functools.partial(
        pl.kernel,
        out_type=jax.ShapeDtypeStruct((BATCH, SEQ, OUTPUT_DIM), jnp.float32),
        mesh=mesh,
        scratch_types=[
            pltpu.VMEM((rows_per_w,), jnp.int32),
            pltpu.VMEM((SEQ, OUTPUT_DIM), jnp.float32),
            [pltpu.VMEM((CHUNK, PAD_DIM), jnp.float32) for _ in range(NBUF)],
            [pltpu.VMEM((CHUNK, OUTPUT_DIM), jnp.float32) for _ in range(NBUF)],
            [pltpu.SemaphoreType.DMA for _ in range(NBUF)],
            [pltpu.SemaphoreType.DMA for _ in range(NBUF)],
        ],
        compiler_params=pltpu.CompilerParams(use_tc_tiling_on_sc=True),
    )
    def emb_kernel(idx_hbm, table_hbm, pos_hbm, out_hbm, idx_v, pos_v,
                   gbufs, sbufs, gsems, ssems):
        wid = lax.axis_index("s") * nc + lax.axis_index("c")
        pltpu.sync_copy(idx_hbm.at[wid], idx_v)
        pltpu.sync_copy(pos_hbm, pos_v)
        base = wid * seqs_per_w

        def gather_copy(ci, gb, gsem):
            return pltpu.make_async_copy(
                table_hbm.at[idx_v.at[pl.ds(ci * CHUNK, CHUNK)]], gb, gsem)

        def store_copy(ci, sb, ssem):
            return pltpu.make_async_copy(
                sb, out_hbm.at[base + ci // cps,
                               pl.ds((ci % cps) * CHUNK, CHUNK)], ssem)

        for b in range(NBUF):
            gather_copy(b, gbufs[b], gsems[b]).start()

        def body(j, carry):
            c0 = NBUF * j
            for b in range(NBUF):
                ci = c0 + b
                gb, sb, gsem, ssem = gbufs[b], sbufs[b], gsems[b], ssems[b]
                gather_copy(ci, gb, gsem).wait()

                @pl.when(ci >= NBUF)
                def _():
                    store_copy(ci - NBUF, sb, ssem).wait()

                poff = (ci % cps) * CHUNK

                @functools.partial(plsc.parallel_loop, 0, CHUNK, unroll=4)
                def _(r):
                    for c in range(OUTPUT_DIM // 16):
                        sl = pl.ds(c * 16, 16)
                        sb[r, sl] = gb[r, sl] * SCALE + pos_v[poff + r, sl]

                store_copy(ci, sb, ssem).start()

                @pl.when(ci + NBUF < chunks_per_w)
                def _():
                    gather_copy(ci + NBUF, gb, gsem).start()
            return carry

        lax.fori_loop(0, chunks_per_w // NBUF, body, 0)
        for b in range(NBUF):
            store_copy(chunks_per_w - NBUF + b, sbufs[b], ssems[b]).wait()

    return emb_kernel


def kernel(x, table):
    info = plsc.get_sparse_core_info()
    nc, ns = info.num_cores, info.num_subcores
    nw = nc * ns
    idx = x.reshape(nw, (BATCH * SEQ) // nw)
    table_p = jnp.pad(table, ((0, 0), (0, PAD_DIM - OUTPUT_DIM)))
    pos = jnp.asarray(_POS)
    return _build_kernel(nc, ns)(idx, table_p, pos)
